# baseline (speedup 1.0000x reference)
import functools

import jax
import jax.numpy as jnp
from jax import lax
from jax.experimental import pallas as pl
from jax.experimental.pallas import tpu as pltpu
from jax.experimental.pallas import tpu_sc as plsc

_NC = 2
_NS = 16
_NW = _NC * _NS

_CHUNK = 128


def _make_gather(num_rows, dim, n_chunks):
    per_w = n_chunks * _CHUNK
    total = _NW * per_w
    mesh = plsc.VectorSubcoreMesh(core_axis_name="c", subcore_axis_name="s")

    @functools.partial(
        pl.kernel,
        out_type=jax.ShapeDtypeStruct((total, dim), jnp.float32),
        mesh=mesh,
        scratch_types=[
            pltpu.VMEM((n_chunks, _CHUNK), jnp.int32),
            pltpu.VMEM((_CHUNK, dim), jnp.float32),
            pltpu.SemaphoreType.DMA,
        ],
        compiler_params=pltpu.CompilerParams(use_tc_tiling_on_sc=False),
    )
    def gather(idx_hbm, table_hbm, out_hbm, idx_v, rows_v, sem):
        wid = lax.axis_index("s") * _NC + lax.axis_index("c")
        base = wid * per_w
        pltpu.sync_copy(idx_hbm.at[wid], idx_v)

        @pl.loop(0, n_chunks)
        def _(g):
            pltpu.async_copy(table_hbm.at[idx_v.at[g]], rows_v, sem).wait()
            pltpu.sync_copy(rows_v, out_hbm.at[pl.ds(base + g * _CHUNK, _CHUNK)])

    return gather


def kernel(text, seq_len, text_embed_weight):
    b, nt = text.shape
    num_rows, dim = text_embed_weight.shape
    total = b * nt
    n_chunks = total // (_NW * _CHUNK)
    idx = text.astype(jnp.int32).reshape(_NW, n_chunks, _CHUNK)
    gather = _make_gather(num_rows, dim, n_chunks)
    out = gather(idx, text_embed_weight)
    return out.reshape(b, nt, dim)


# speedup vs baseline: 1.1180x; 1.1180x over previous
import functools

import jax
import jax.numpy as jnp
from jax import lax
from jax.experimental import pallas as pl
from jax.experimental.pallas import tpu as pltpu
from jax.experimental.pallas import tpu_sc as plsc

_NC = 2
_NS = 16
_NW = _NC * _NS

_CHUNK = 128


_NBUF = 8


def _make_gather(num_rows, dim, n_chunks):
    per_w = n_chunks * _CHUNK
    total = _NW * per_w
    assert n_chunks % _NBUF == 0
    nsteps = n_chunks // _NBUF
    mesh = plsc.VectorSubcoreMesh(core_axis_name="c", subcore_axis_name="s")

    @functools.partial(
        pl.kernel,
        out_type=jax.ShapeDtypeStruct((total, dim), jnp.float32),
        mesh=mesh,
        scratch_types=[
            pltpu.VMEM((n_chunks, _CHUNK), jnp.int32),
            pltpu.VMEM((_NBUF * _CHUNK, dim), jnp.float32),
            [pltpu.SemaphoreType.DMA] * _NBUF,
        ],
        compiler_params=pltpu.CompilerParams(use_tc_tiling_on_sc=False),
    )
    def gather(idx_hbm, table_hbm, out_hbm, idx_v, rows_v, sems):
        wid = lax.axis_index("s") * _NC + lax.axis_index("c")
        base = wid * per_w
        pltpu.sync_copy(idx_hbm.at[wid], idx_v)

        def buf(b):
            return rows_v.at[pl.ds(b * _CHUNK, _CHUNK)]

        for b in range(_NBUF):
            pltpu.async_copy(table_hbm.at[idx_v.at[b]], buf(b), sems[b])

        @pl.loop(0, nsteps - 1)
        def _(s):
            for b in range(_NBUF):
                c = s * _NBUF + b
                pltpu.make_async_copy(out_hbm.at[pl.ds(0, _CHUNK)], buf(b),
                                      sems[b]).wait()
                pltpu.sync_copy(buf(b), out_hbm.at[pl.ds(base + c * _CHUNK,
                                                         _CHUNK)])
                pltpu.async_copy(table_hbm.at[idx_v.at[c + _NBUF]], buf(b),
                                 sems[b])

        for b in range(_NBUF):
            c = (nsteps - 1) * _NBUF + b
            pltpu.make_async_copy(out_hbm.at[pl.ds(0, _CHUNK)], buf(b),
                                  sems[b]).wait()
            pltpu.sync_copy(buf(b), out_hbm.at[pl.ds(base + c * _CHUNK,
                                                     _CHUNK)])

    return gather


def kernel(text, seq_len, text_embed_weight):
    b, nt = text.shape
    num_rows, dim = text_embed_weight.shape
    total = b * nt
    n_chunks = total // (_NW * _CHUNK)
    idx = text.astype(jnp.int32).reshape(_NW, n_chunks, _CHUNK)
    gather = _make_gather(num_rows, dim, n_chunks)
    out = gather(idx, text_embed_weight)
    return out.reshape(b, nt, dim)
